# baseline (speedup 1.0000x reference)
import functools

import jax
import jax.numpy as jnp
from jax import lax
from jax.experimental import pallas as pl
from jax.experimental.pallas import tpu as pltpu
from jax.experimental.pallas import tpu_sc as plsc

_N_TOKENS = 32768
_D = 768
_E = 64
_TM = 4096

_NC, _NS, _L = 2, 16, 16
_NW = _NC * _NS
_U = 4


def _fused_body(x_ref, w_ref, b_ref, scores_ref, ts_ref, ti_ref):
    x = x_ref[...]
    w = w_ref[...]
    logits = lax.dot_general(
        w, x, (((1,), (1,)), ((), ())), preferred_element_type=jnp.float32
    )
    logits = logits + b_ref[...]
    m = jnp.max(logits, axis=0, keepdims=True)
    e = jnp.exp(logits - m)
    s = jnp.sum(e, axis=0, keepdims=True)
    scores = e * (1.0 / s)
    scores_ref[...] = scores

    iota = lax.broadcasted_iota(jnp.int32, scores.shape, 0)
    m1 = jnp.max(scores, axis=0, keepdims=True)
    i1 = jnp.min(jnp.where(scores == m1, iota, _E), axis=0, keepdims=True)
    masked = jnp.where(iota == i1, -jnp.inf, scores)
    m2 = jnp.max(masked, axis=0, keepdims=True)
    i2 = jnp.min(jnp.where(masked == m2, iota, _E), axis=0, keepdims=True)
    ts_ref[...] = jnp.concatenate([m1, m2], axis=0)
    ti_ref[...] = jnp.concatenate([i1, i2], axis=0)


def _fused_router(x, W, b):
    scores_t, ts_t, ti_t = pl.pallas_call(
        _fused_body,
        grid=(_N_TOKENS // _TM,),
        in_specs=[
            pl.BlockSpec((_TM, _D), lambda i: (i, 0)),
            pl.BlockSpec((_E, _D), lambda i: (0, 0)),
            pl.BlockSpec((_E, 1), lambda i: (0, 0)),
        ],
        out_specs=[
            pl.BlockSpec((_E, _TM), lambda i: (0, i)),
            pl.BlockSpec((2, _TM), lambda i: (0, i)),
            pl.BlockSpec((2, _TM), lambda i: (0, i)),
        ],
        out_shape=[
            jax.ShapeDtypeStruct((_E, _N_TOKENS), jnp.float32),
            jax.ShapeDtypeStruct((2, _N_TOKENS), jnp.float32),
            jax.ShapeDtypeStruct((2, _N_TOKENS), jnp.int32),
        ],
    )(x, W, b.reshape(_E, 1))
    return scores_t.T, ts_t.T, ti_t.T


def _make_sc_topk(T):
    tpw = T // _NW
    groups = tpw // _L
    mesh = plsc.VectorSubcoreMesh(
        core_axis_name="c", subcore_axis_name="s",
        num_cores=_NC, num_subcores=_NS,
    )

    @functools.partial(
        pl.kernel,
        out_type=[
            jax.ShapeDtypeStruct((T,), jnp.float32),
            jax.ShapeDtypeStruct((T,), jnp.float32),
            jax.ShapeDtypeStruct((T,), jnp.int32),
            jax.ShapeDtypeStruct((T,), jnp.int32),
        ],
        mesh=mesh,
        compiler_params=pltpu.CompilerParams(needs_layout_passes=False),
        scratch_types=[
            pltpu.VMEM((tpw * (_E + 1),), jnp.float32),
            pltpu.VMEM((tpw,), jnp.float32),
            pltpu.VMEM((tpw,), jnp.float32),
            pltpu.VMEM((tpw,), jnp.int32),
            pltpu.VMEM((tpw,), jnp.int32),
        ],
    )
    def sc_topk(scores_hbm, s1_hbm, s2_hbm, i1_hbm, i2_hbm,
                sc_v, s1_v, s2_v, i1_v, i2_v):
        wid = lax.axis_index("s") * _NC + lax.axis_index("c")
        base = wid * tpw
        pltpu.sync_copy(scores_hbm.at[pl.ds(base * _E, tpw * _E)],
                        sc_v.at[pl.ds(0, tpw * _E)])

        lane = lax.broadcasted_iota(jnp.int32, (_L,), 0)

        def respread(r_up, carry):
            r = tpw - 1 - r_up
            for c in (3, 2, 1, 0):
                v = sc_v[pl.ds(r * _E + c * _L, _L)]
                sc_v[pl.ds(r * (_E + 1) + c * _L, _L)] = v
            return carry

        lax.fori_loop(1, tpw, respread, 0, unroll=4)

        def group(g, carry):
            rows, m1s, i1s, m2s, i2s = [], [], [], [], []
            zero = jnp.zeros((_L,), jnp.int32)
            for j in range(_U):
                row = ((g * _U + j) * _L + lane) * (_E + 1)
                rows.append(row)
                m1s.append(plsc.load_gather(sc_v, [row]))
                i1s.append(zero)
                m2s.append(jnp.full((_L,), -jnp.inf, jnp.float32))
                i2s.append(zero)
            for e in range(1, _E):
                ev = jnp.full((_L,), e, jnp.int32)
                for j in range(_U):
                    v = plsc.load_gather(sc_v, [rows[j] + e])
                    gt1 = v > m1s[j]
                    gt2 = v > m2s[j]
                    i2s[j] = jnp.where(gt1, i1s[j],
                                       jnp.where(gt2, ev, i2s[j]))
                    m2s[j] = jnp.maximum(m2s[j], jnp.minimum(m1s[j], v))
                    i1s[j] = jnp.where(gt1, ev, i1s[j])
                    m1s[j] = jnp.maximum(m1s[j], v)
            for j in range(_U):
                off = (g * _U + j) * _L
                s1_v[pl.ds(off, _L)] = m1s[j]
                s2_v[pl.ds(off, _L)] = m2s[j]
                i1_v[pl.ds(off, _L)] = i1s[j]
                i2_v[pl.ds(off, _L)] = i2s[j]
            return carry

        lax.fori_loop(0, groups // _U, group, 0)

        pltpu.sync_copy(s1_v, s1_hbm.at[pl.ds(base, tpw)])
        pltpu.sync_copy(s2_v, s2_hbm.at[pl.ds(base, tpw)])
        pltpu.sync_copy(i1_v, i1_hbm.at[pl.ds(base, tpw)])
        pltpu.sync_copy(i2_v, i2_hbm.at[pl.ds(base, tpw)])

    return sc_topk


def _dense_body_t(x_ref, w_ref, b_ref, scores_ref):
    x = x_ref[...]
    w = w_ref[...]
    logits = lax.dot_general(
        w, x, (((1,), (1,)), ((), ())), preferred_element_type=jnp.float32
    )
    logits = logits + b_ref[...]
    m = jnp.max(logits, axis=0, keepdims=True)
    e = jnp.exp(logits - m)
    s = jnp.sum(e, axis=0, keepdims=True)
    scores_ref[...] = e * (1.0 / s)


def _dense_scores_t(x, W, b):
    scores_t = pl.pallas_call(
        _dense_body_t,
        grid=(_N_TOKENS // _TM,),
        in_specs=[
            pl.BlockSpec((_TM, _D), lambda i: (i, 0)),
            pl.BlockSpec((_E, _D), lambda i: (0, 0)),
            pl.BlockSpec((_E, 1), lambda i: (0, 0)),
        ],
        out_specs=pl.BlockSpec((_E, _TM), lambda i: (0, i)),
        out_shape=jax.ShapeDtypeStruct((_E, _N_TOKENS), jnp.float32),
    )(x, W, b.reshape(_E, 1))
    return scores_t.T


def kernel(x, W, b):
    scores = _dense_scores_t(x, W, b)
    s1, s2, i1, i2 = _make_sc_topk(_N_TOKENS)(scores.reshape(-1))
    ts = jnp.stack([s1, s2], axis=-1)
    ti = jnp.stack([i1, i2], axis=-1)
    return ts, ti, scores


# speedup vs baseline: 2.8324x; 2.8324x over previous
import jax
import jax.numpy as jnp
from jax import lax
from jax.experimental import pallas as pl

_N_TOKENS = 32768
_D = 768
_E = 64
_TM = 4096


def _fused_body(x_ref, w_ref, b_ref, scores_ref, ts_ref, ti_ref):
    x = x_ref[...]
    w = w_ref[...]
    logits = lax.dot_general(
        w, x, (((1,), (1,)), ((), ())), preferred_element_type=jnp.float32
    )
    logits = logits + b_ref[...]
    m = jnp.max(logits, axis=0, keepdims=True)
    e = jnp.exp(logits - m)
    s = jnp.sum(e, axis=0, keepdims=True)
    scores = e * (1.0 / s)
    scores_ref[...] = scores

    iota = lax.broadcasted_iota(jnp.int32, scores.shape, 0)
    m1 = jnp.max(scores, axis=0, keepdims=True)
    i1 = jnp.min(jnp.where(scores == m1, iota, _E), axis=0, keepdims=True)
    masked = jnp.where(iota == i1, -jnp.inf, scores)
    m2 = jnp.max(masked, axis=0, keepdims=True)
    i2 = jnp.min(jnp.where(masked == m2, iota, _E), axis=0, keepdims=True)
    ts_ref[...] = jnp.concatenate([m1, m2], axis=0)
    ti_ref[...] = jnp.concatenate([i1, i2], axis=0)


def kernel(x, W, b):
    scores_t, ts_t, ti_t = pl.pallas_call(
        _fused_body,
        grid=(_N_TOKENS // _TM,),
        in_specs=[
            pl.BlockSpec((_TM, _D), lambda i: (i, 0)),
            pl.BlockSpec((_E, _D), lambda i: (0, 0)),
            pl.BlockSpec((_E, 1), lambda i: (0, 0)),
        ],
        out_specs=[
            pl.BlockSpec((_E, _TM), lambda i: (0, i)),
            pl.BlockSpec((2, _TM), lambda i: (0, i)),
            pl.BlockSpec((2, _TM), lambda i: (0, i)),
        ],
        out_shape=[
            jax.ShapeDtypeStruct((_E, _N_TOKENS), jnp.float32),
            jax.ShapeDtypeStruct((2, _N_TOKENS), jnp.float32),
            jax.ShapeDtypeStruct((2, _N_TOKENS), jnp.int32),
        ],
    )(x, W, b.reshape(_E, 1))
    return ts_t.T, ti_t.T, scores_t.T
